# baseline (speedup 1.0000x reference)
import jax
import jax.numpy as jnp
from jax.experimental import pallas as pl
from jax.experimental.pallas import tpu as pltpu


def kernel(masks, idx):
    M, D = masks.shape
    B = idx.shape[0]
    R, C = D // 128, 128

    def body(idx_ref, masks_ref, out_ref, sem):
        copies = [
            pltpu.make_async_copy(
                masks_ref.at[idx_ref[b]], out_ref.at[b], sem
            )
            for b in range(B)
        ]
        for c in copies:
            c.start()
        for c in copies:
            c.wait()

    grid_spec = pltpu.PrefetchScalarGridSpec(
        num_scalar_prefetch=1,
        grid=(1,),
        in_specs=[pl.BlockSpec(memory_space=pl.ANY)],
        out_specs=pl.BlockSpec(memory_space=pl.ANY),
        scratch_shapes=[pltpu.SemaphoreType.DMA],
    )
    out = pl.pallas_call(
        body,
        grid_spec=grid_spec,
        out_shape=jax.ShapeDtypeStruct((B, R, C), jnp.uint8),
    )(idx, masks.astype(jnp.uint8).reshape(M, R, C))
    return out.reshape(B, D) != 0


# speedup vs baseline: 1.9019x; 1.9019x over previous
import functools

import jax
import jax.numpy as jnp
from jax import lax
from jax.experimental import pallas as pl
from jax.experimental.pallas import tpu as pltpu
from jax.experimental.pallas import tpu_sc as plsc

_INFO = plsc.get_sparse_core_info()
_NC = _INFO.num_cores
_NS = _INFO.num_subcores
_NW = _NC * _NS


def kernel(masks, idx):
    M, D = masks.shape
    B = idx.shape[0]
    bpw = B // _NW
    half = bpw // 2

    mesh = plsc.VectorSubcoreMesh(core_axis_name="c", subcore_axis_name="s")

    @functools.partial(
        pl.kernel,
        mesh=mesh,
        out_type=jax.ShapeDtypeStruct((B, D), masks.dtype),
        scratch_types=[
            pltpu.VMEM((half,), jnp.int32),
            pltpu.VMEM((half,), jnp.int32),
            pltpu.VMEM((half, D), masks.dtype),
            pltpu.VMEM((half, D), masks.dtype),
            pltpu.SemaphoreType.DMA,
            pltpu.SemaphoreType.DMA,
            pltpu.SemaphoreType.DMA,
            pltpu.SemaphoreType.DMA,
        ],
    )
    def run(masks_hbm, idx_hbm, out_hbm, ia, ib, buf0, buf1, g0, g1, w0, w1):
        wid = lax.axis_index("s") * _NC + lax.axis_index("c")
        base = wid * bpw
        ca = pltpu.async_copy(idx_hbm.at[2 * wid], ia, g0)
        cb = pltpu.async_copy(idx_hbm.at[2 * wid + 1], ib, g1)
        ca.wait()
        cg0 = pltpu.async_copy(masks_hbm.at[ia], buf0, g0)
        cb.wait()
        cg1 = pltpu.async_copy(masks_hbm.at[ib], buf1, g1)
        cg0.wait()
        cw0 = pltpu.async_copy(buf0, out_hbm.at[pl.ds(base, half)], w0)
        cg1.wait()
        cw1 = pltpu.async_copy(buf1, out_hbm.at[pl.ds(base + half, half)], w1)
        cw0.wait()
        cw1.wait()

    return run(masks, idx.reshape(2 * _NW, half))
